# baseline (speedup 1.0000x reference)
import jax
import jax.numpy as jnp
from jax.experimental import pallas as pl
from jax.experimental.pallas import tpu as pltpu

N = 8192
D = 1024
BN = 1024


def _fused_kernel(xs_ref, wd_ref, bd_ref, wl_ref, bl_ref, wr_ref, br_ref,
                  out_ref, wl16_ref, wr16_ref):
    @pl.when(pl.program_id(0) == 0)
    def _cast_weights():
        wl16_ref[...] = wl_ref[...].astype(jnp.bfloat16)
        wr16_ref[...] = wr_ref[...].astype(jnp.bfloat16)

    SB = 256
    for s in range(BN // SB):
        x = xs_ref[pl.ds(s * SB, SB), :]
        dec = jnp.dot(x, wd_ref[...],
                      preferred_element_type=jnp.float32) + bd_ref[0, 0]
        xb = x.astype(jnp.bfloat16)
        l = jnp.dot(xb, wl16_ref[...],
                    preferred_element_type=jnp.float32) + bl_ref[...]
        r = jnp.dot(xb, wr16_ref[...],
                    preferred_element_type=jnp.float32) + br_ref[...]
        out_ref[pl.ds(s * SB, SB), :] = jnp.where(dec > 0.0, r, l)


def kernel(xs, w_dec, b_dec, W_left, b_left, W_right, b_right):
    wd = w_dec.reshape(D, 1)
    bd = b_dec.reshape(1, 1)
    bl = b_left.reshape(1, D)
    br = b_right.reshape(1, D)
    grid = (N // BN,)
    return pl.pallas_call(
        _fused_kernel,
        grid=grid,
        in_specs=[
            pl.BlockSpec((BN, D), lambda i: (i, 0)),
            pl.BlockSpec((D, 1), lambda i: (0, 0)),
            pl.BlockSpec((1, 1), lambda i: (0, 0)),
            pl.BlockSpec((D, D), lambda i: (0, 0)),
            pl.BlockSpec((1, D), lambda i: (0, 0)),
            pl.BlockSpec((D, D), lambda i: (0, 0)),
            pl.BlockSpec((1, D), lambda i: (0, 0)),
        ],
        out_specs=pl.BlockSpec((BN, D), lambda i: (i, 0)),
        out_shape=jax.ShapeDtypeStruct((N, D), jnp.float32),
        scratch_shapes=[
            pltpu.VMEM((D, D), jnp.bfloat16),
            pltpu.VMEM((D, D), jnp.bfloat16),
        ],
    )(xs, wd, bd, W_left, bl, W_right, br)


# speedup vs baseline: 1.1338x; 1.1338x over previous
import jax
import jax.numpy as jnp
from jax.experimental import pallas as pl
from jax.experimental.pallas import tpu as pltpu

N = 8192
D = 1024
BN = 1024


def _fused_kernel(xs_ref, wd_ref, bd_ref, wl_ref, bl_ref, wr_ref, br_ref,
                  out_ref, wl16_ref, wr16_ref):
    @pl.when(pl.program_id(0) == 0)
    def _cast_weights():
        wl16_ref[...] = wl_ref[...].astype(jnp.bfloat16)
        wr16_ref[...] = wr_ref[...].astype(jnp.bfloat16)

    x = xs_ref[...]
    dec = jnp.sum(x * wd_ref[...], axis=1, keepdims=True) + bd_ref[0, 0]
    xb = x.astype(jnp.bfloat16)
    l = jnp.dot(xb, wl16_ref[...], preferred_element_type=jnp.float32) + bl_ref[...]
    r = jnp.dot(xb, wr16_ref[...], preferred_element_type=jnp.float32) + br_ref[...]
    out_ref[...] = jnp.where(dec > 0.0, r, l)


def kernel(xs, w_dec, b_dec, W_left, b_left, W_right, b_right):
    wd = w_dec.reshape(1, D)
    bd = b_dec.reshape(1, 1)
    bl = b_left.reshape(1, D)
    br = b_right.reshape(1, D)
    grid = (N // BN,)
    return pl.pallas_call(
        _fused_kernel,
        grid=grid,
        in_specs=[
            pl.BlockSpec((BN, D), lambda i: (i, 0)),
            pl.BlockSpec((1, D), lambda i: (0, 0)),
            pl.BlockSpec((1, 1), lambda i: (0, 0)),
            pl.BlockSpec((D, D), lambda i: (0, 0)),
            pl.BlockSpec((1, D), lambda i: (0, 0)),
            pl.BlockSpec((D, D), lambda i: (0, 0)),
            pl.BlockSpec((1, D), lambda i: (0, 0)),
        ],
        out_specs=pl.BlockSpec((BN, D), lambda i: (i, 0)),
        out_shape=jax.ShapeDtypeStruct((N, D), jnp.float32),
        scratch_shapes=[
            pltpu.VMEM((D, D), jnp.bfloat16),
            pltpu.VMEM((D, D), jnp.bfloat16),
        ],
    )(xs, wd, bd, W_left, bl, W_right, br)
